# baseline (speedup 1.0000x reference)
import functools

import jax
import jax.numpy as jnp
from jax import lax
from jax.experimental import pallas as pl
from jax.experimental.pallas import tpu as pltpu
from jax.experimental.pallas import tpu_sc as plsc

N_ROWS = 320000
D = 128
N_CLS = 1000
PAD_CLS = 1024
N_CORES = 2
N_SUB = 16
N_WORK = N_CORES * N_SUB
ROWS_PER_W = N_ROWS // N_WORK
CHUNK = 200
SUB = 40
NSUB = CHUNK // SUB
NRING = 4
CHUNKS_PER_W = ROWS_PER_W // CHUNK
LPAD = 208
ROWS_PER_TILE = PAD_CLS // N_SUB


def _make_main():
  mesh = plsc.VectorSubcoreMesh(core_axis_name="c", subcore_axis_name="s")

  @functools.partial(
      pl.kernel,
      mesh=mesh,
      out_type=[
          jax.ShapeDtypeStruct((N_CORES * PAD_CLS, D), jnp.float32),
          jax.ShapeDtypeStruct((N_WORK, PAD_CLS), jnp.float32),
      ],
      scratch_types=[
          pltpu.VMEM((NRING, CHUNK, D), jnp.float32),
          pltpu.VMEM((NRING, NSUB, SUB), jnp.int32),
          pltpu.VMEM((LPAD,), jnp.int32),
          pltpu.VMEM((LPAD,), jnp.int32),
          pltpu.VMEM((LPAD,), jnp.int32),
          pltpu.VMEM((LPAD,), jnp.int32),
          pltpu.VMEM((PAD_CLS,), jnp.float32),
          pltpu.VMEM_SHARED((PAD_CLS, D), jnp.float32),
          pltpu.SemaphoreType.DMA,
          pltpu.SemaphoreType.DMA,
          pltpu.SemaphoreType.DMA,
          pltpu.SemaphoreType.DMA,
          pltpu.SemaphoreType.DMA,
          pltpu.SemaphoreType.DMA,
          pltpu.SemaphoreType.DMA,
          pltpu.SemaphoreType.DMA,
          pltpu.SemaphoreType.DMA,
          pltpu.SemaphoreType.DMA,
          pltpu.SemaphoreType.DMA,
          pltpu.SemaphoreType.DMA,
      ],
      compiler_params=pltpu.CompilerParams(needs_layout_passes=False),
  )
  def main(feat_hbm, lbl_hbm, zf_hbm,
           sums_hbm, cnts_hbm,
           fbuf, libuf, lfa, lfb, lfc, lfd, cnt_local, acc_sh,
           gf0, gf1, gf2, gf3, gl0, gl1, gl2, gl3, ss0, ss1, ss2, ss3):
    c = lax.axis_index("c")
    s = lax.axis_index("s")
    wid = s * N_CORES + c
    base = wid * ROWS_PER_W
    gf = (gf0, gf1, gf2, gf3)
    gl = (gl0, gl1, gl2, gl3)
    ss = (ss0, ss1, ss2, ss3)
    lf = (lfa, lfb, lfc, lfd)

    pltpu.sync_copy(zf_hbm, acc_sh.at[pl.ds(s * ROWS_PER_TILE, ROWS_PER_TILE)])

    def zbody(i, carry):
      cnt_local[pl.ds(i * 16, 16)] = jnp.zeros((16,), jnp.float32)
      return carry

    lax.fori_loop(0, PAD_CLS // 16, zbody, 0)
    plsc.subcore_barrier()

    ones16 = jnp.ones((16,), jnp.float32)
    tail_mask = lax.iota(jnp.int32, 16) < (CHUNK - (CHUNK // 16) * 16)

    def start_gather(j, r):
      off = base + j * CHUNK
      pltpu.async_copy(feat_hbm.at[pl.ds(off, CHUNK)], fbuf.at[r], gf[r])
      for k in range(NSUB):
        pltpu.async_copy(lbl_hbm.at[pl.ds(off + k * SUB, SUB)],
                         libuf.at[r, k], gl[r])
      pltpu.async_copy(lbl_hbm.at[pl.ds(off, LPAD)], lf[r], gl[r])

    def wait_gather(r):
      pltpu.make_async_copy(feat_hbm.at[pl.ds(0, CHUNK)], fbuf.at[r],
                            gf[r]).wait()
      for k in range(NSUB):
        pltpu.make_async_copy(lbl_hbm.at[pl.ds(0, SUB)], libuf.at[r, k],
                              gl[r]).wait()
      pltpu.make_async_copy(lbl_hbm.at[pl.ds(0, LPAD)], lf[r], gl[r]).wait()

    def start_scatter(r):
      for k in range(NSUB):
        pltpu.async_copy(fbuf.at[r, pl.ds(k * SUB, SUB)],
                         acc_sh.at[libuf.at[r, k]], ss[r], add=True)

    def wait_scatter(r):
      for k in range(NSUB):
        pltpu.make_async_copy(fbuf.at[r, pl.ds(k * SUB, SUB)],
                              acc_sh.at[libuf.at[r, k]], ss[r]).wait()

    def counts(r):
      nfull = CHUNK // 16
      for k in range(nfull):
        lv = lf[r][pl.ds(k * 16, 16)]
        plsc.addupdate_scatter(cnt_local, [lv], ones16)
      lv = lf[r][pl.ds(nfull * 16, 16)]
      plsc.addupdate_scatter(cnt_local, [lv], ones16, mask=tail_mask)

    for r in range(2):
      start_gather(r, r)

    def body(t, carry):
      for r in range(NRING):
        j = NRING * t + r
        wait_gather(r)

        @pl.when(j >= 2)
        def _():
          wait_scatter((r + 2) % NRING)

        @pl.when(j + 2 < CHUNKS_PER_W)
        def _():
          start_gather(j + 2, (r + 2) % NRING)

        start_scatter(r)
        counts(r)
      return carry

    lax.fori_loop(0, CHUNKS_PER_W // NRING, body, 0)

    wait_gather(0)
    wait_scatter(2)
    start_scatter(0)
    counts(0)
    wait_gather(1)
    wait_scatter(3)
    start_scatter(1)
    counts(1)
    wait_scatter(0)
    wait_scatter(1)

    plsc.subcore_barrier()
    outbase = c * PAD_CLS + s * ROWS_PER_TILE
    pltpu.sync_copy(acc_sh.at[pl.ds(s * ROWS_PER_TILE, ROWS_PER_TILE)],
                    sums_hbm.at[pl.ds(outbase, ROWS_PER_TILE)])
    pltpu.sync_copy(cnt_local, cnts_hbm.at[wid])

  return main


def _combine_body(s_ref, c_ref, o_ref):
  s0 = s_ref[0:PAD_CLS, :]
  s1 = s_ref[PAD_CLS:2 * PAD_CLS, :]
  cnt = jnp.sum(c_ref[...], axis=0)
  o_ref[...] = (s0 + s1) / cnt[:, None]


@jax.jit
def _run(support_features, support_labels):
  labels = support_labels.astype(jnp.int32)
  labels_p = jnp.pad(labels, (0, LPAD - CHUNK))
  zf = jnp.zeros((ROWS_PER_TILE, D), jnp.float32)
  sums, cnts = _make_main()(support_features, labels_p, zf)
  out = pl.pallas_call(
      _combine_body,
      out_shape=jax.ShapeDtypeStruct((PAD_CLS, D), jnp.float32),
  )(sums, cnts)
  return out[:N_CLS]


def kernel(support_features, support_labels, n_way):
  return _run(support_features, support_labels)


# speedup vs baseline: 1.0432x; 1.0432x over previous
import functools

import jax
import jax.numpy as jnp
from jax import lax
from jax.experimental import pallas as pl
from jax.experimental.pallas import tpu as pltpu
from jax.experimental.pallas import tpu_sc as plsc

N_ROWS = 320000
D = 128
N_CLS = 1000
PAD_CLS = 1024
N_CORES = 2
N_SUB = 16
N_WORK = N_CORES * N_SUB
ROWS_PER_W = N_ROWS // N_WORK
CHUNK = 200
SUB = 40
NSUB = CHUNK // SUB
NRING = 4
CHUNKS_PER_W = ROWS_PER_W // CHUNK
LPAD = 208
ROWS_PER_TILE = PAD_CLS // N_SUB


def _make_main():
  mesh = plsc.VectorSubcoreMesh(core_axis_name="c", subcore_axis_name="s")

  @functools.partial(
      pl.kernel,
      mesh=mesh,
      out_type=[
          jax.ShapeDtypeStruct((N_CORES * PAD_CLS, D), jnp.float32),
          jax.ShapeDtypeStruct((N_WORK, PAD_CLS), jnp.float32),
      ],
      scratch_types=[
          pltpu.VMEM((NRING, CHUNK, D), jnp.float32),
          pltpu.VMEM((NRING, NSUB, SUB), jnp.int32),
          pltpu.VMEM((LPAD,), jnp.int32),
          pltpu.VMEM((LPAD,), jnp.int32),
          pltpu.VMEM((LPAD,), jnp.int32),
          pltpu.VMEM((LPAD,), jnp.int32),
          pltpu.VMEM((PAD_CLS,), jnp.float32),
          pltpu.VMEM_SHARED((PAD_CLS, D), jnp.float32),
          pltpu.SemaphoreType.DMA,
          pltpu.SemaphoreType.DMA,
          pltpu.SemaphoreType.DMA,
          pltpu.SemaphoreType.DMA,
          pltpu.SemaphoreType.DMA,
          pltpu.SemaphoreType.DMA,
          pltpu.SemaphoreType.DMA,
          pltpu.SemaphoreType.DMA,
          pltpu.SemaphoreType.DMA,
          pltpu.SemaphoreType.DMA,
          pltpu.SemaphoreType.DMA,
          pltpu.SemaphoreType.DMA,
      ],
      compiler_params=pltpu.CompilerParams(needs_layout_passes=False),
  )
  def main(feat_hbm, lbl_hbm, zf_hbm,
           sums_hbm, cnts_hbm,
           fbuf, libuf, lfa, lfb, lfc, lfd, cnt_local, acc_sh,
           gf0, gf1, gf2, gf3, gl0, gl1, gl2, gl3, ss0, ss1, ss2, ss3):
    c = lax.axis_index("c")
    s = lax.axis_index("s")
    wid = s * N_CORES + c
    base = wid * ROWS_PER_W
    gf = (gf0, gf1, gf2, gf3)
    gl = (gl0, gl1, gl2, gl3)
    ss = (ss0, ss1, ss2, ss3)
    lf = (lfa, lfb, lfc, lfd)

    pltpu.sync_copy(zf_hbm, acc_sh.at[pl.ds(s * ROWS_PER_TILE, ROWS_PER_TILE)])

    def zbody(i, carry):
      cnt_local[pl.ds(i * 16, 16)] = jnp.zeros((16,), jnp.float32)
      return carry

    lax.fori_loop(0, PAD_CLS // 16, zbody, 0)
    plsc.subcore_barrier()

    ones16 = jnp.ones((16,), jnp.float32)
    tail_mask = lax.iota(jnp.int32, 16) < (CHUNK - (CHUNK // 16) * 16)

    def start_gather(j, r):
      off = base + j * CHUNK
      pltpu.async_copy(feat_hbm.at[pl.ds(off, CHUNK)], fbuf.at[r], gf[r])
      for k in range(NSUB):
        pltpu.async_copy(lbl_hbm.at[pl.ds(off + k * SUB, SUB)],
                         libuf.at[r, k], gl[r])
      pltpu.async_copy(lbl_hbm.at[pl.ds(off, LPAD)], lf[r], gl[r])

    def wait_gather(r):
      pltpu.make_async_copy(feat_hbm.at[pl.ds(0, CHUNK)], fbuf.at[r],
                            gf[r]).wait()
      for k in range(NSUB):
        pltpu.make_async_copy(lbl_hbm.at[pl.ds(0, SUB)], libuf.at[r, k],
                              gl[r]).wait()
      pltpu.make_async_copy(lbl_hbm.at[pl.ds(0, LPAD)], lf[r], gl[r]).wait()

    def start_scatter(r):
      for k in range(NSUB):
        pltpu.async_copy(fbuf.at[r, pl.ds(k * SUB, SUB)],
                         acc_sh.at[libuf.at[r, k]], ss[r], add=True)

    def wait_scatter(r):
      for k in range(NSUB):
        pltpu.make_async_copy(fbuf.at[r, pl.ds(k * SUB, SUB)],
                              acc_sh.at[libuf.at[r, k]], ss[r]).wait()

    def counts(r):
      nfull = CHUNK // 16
      for k in range(nfull):
        lv = lf[r][pl.ds(k * 16, 16)]
        plsc.addupdate_scatter(cnt_local, [lv], ones16)
      lv = lf[r][pl.ds(nfull * 16, 16)]
      plsc.addupdate_scatter(cnt_local, [lv], ones16, mask=tail_mask)

    for r in range(NRING - 1):
      start_gather(r, r)

    def body(t, carry):
      for r in range(NRING):
        j = NRING * t + r
        wait_gather(r)

        @pl.when(j != 0)
        def _():
          wait_scatter((r + NRING - 1) % NRING)

        @pl.when(j + NRING - 1 < CHUNKS_PER_W)
        def _():
          start_gather(j + NRING - 1, (r + NRING - 1) % NRING)

        start_scatter(r)
        counts(r)
      return carry

    lax.fori_loop(0, CHUNKS_PER_W // NRING, body, 0)

    wait_gather(0)
    wait_scatter(NRING - 1)
    start_scatter(0)
    counts(0)
    wait_gather(1)
    start_scatter(1)
    counts(1)
    wait_scatter(0)
    wait_scatter(1)

    plsc.subcore_barrier()
    outbase = c * PAD_CLS + s * ROWS_PER_TILE
    pltpu.sync_copy(acc_sh.at[pl.ds(s * ROWS_PER_TILE, ROWS_PER_TILE)],
                    sums_hbm.at[pl.ds(outbase, ROWS_PER_TILE)])
    pltpu.sync_copy(cnt_local, cnts_hbm.at[wid])

  return main


def _combine_body(s_ref, c_ref, o_ref):
  s0 = s_ref[0:PAD_CLS, :]
  s1 = s_ref[PAD_CLS:2 * PAD_CLS, :]
  cnt = jnp.sum(c_ref[...], axis=0)
  o_ref[...] = (s0 + s1) / cnt[:, None]


@jax.jit
def _run(support_features, support_labels):
  labels = support_labels.astype(jnp.int32)
  labels_p = jnp.pad(labels, (0, LPAD - CHUNK))
  zf = jnp.zeros((ROWS_PER_TILE, D), jnp.float32)
  sums, cnts = _make_main()(support_features, labels_p, zf)
  out = pl.pallas_call(
      _combine_body,
      out_shape=jax.ShapeDtypeStruct((PAD_CLS, D), jnp.float32),
  )(sums, cnts)
  return out[:N_CLS]


def kernel(support_features, support_labels, n_way):
  return _run(support_features, support_labels)
